# baseline (speedup 1.0000x reference)
import functools

import jax
import jax.numpy as jnp
from jax.experimental import pallas as pl

_B = 32768
_NC = 5
_CF = 64
_CD = 64
_SD = 128

_BB = 1024


def _fused_kernel(counts_ref, mask_ref, wmap_ref, bmap_ref, wred_ref,
                  bred_ref, out_ref):
    wm = wmap_ref[...]
    bm = bmap_ref[...]
    acc = jnp.zeros((counts_ref.shape[0], _SD), jnp.float32) + bred_ref[...]
    for n in range(_NC):
        x = counts_ref[:, n, :]
        m = jnp.maximum(
            jnp.dot(x, wm, preferred_element_type=jnp.float32) + bm, 0.0)
        m = m * mask_ref[:, n:n + 1].astype(jnp.float32)
        acc = acc + jnp.dot(m, wred_ref[n * _CD:(n + 1) * _CD, :],
                            preferred_element_type=jnp.float32)
    out_ref[...] = jnp.maximum(acc, 0.0)


@functools.partial(jax.jit, static_argnames=())
def kernel(counts, observed_mask, W_map, b_map, W_red, b_red):
    grid = _B // _BB
    return pl.pallas_call(
        _fused_kernel,
        grid=(grid,),
        in_specs=[
            pl.BlockSpec((_BB, _NC, _CF), lambda i: (i, 0, 0)),
            pl.BlockSpec((_BB, _NC), lambda i: (i, 0)),
            pl.BlockSpec((_CF, _CD), lambda i: (0, 0)),
            pl.BlockSpec((1, _CD), lambda i: (0, 0)),
            pl.BlockSpec((_NC * _CD, _SD), lambda i: (0, 0)),
            pl.BlockSpec((1, _SD), lambda i: (0, 0)),
        ],
        out_specs=pl.BlockSpec((_BB, _SD), lambda i: (i, 0)),
        out_shape=jax.ShapeDtypeStruct((_B, _SD), jnp.float32),
    )(counts, observed_mask, W_map, b_map.reshape(1, _CD),
      W_red, b_red.reshape(1, _SD))


# speedup vs baseline: 1.5668x; 1.5668x over previous
import jax
import jax.numpy as jnp
from jax.experimental import pallas as pl

_B = 32768
_NC = 5
_CF = 64
_CD = 64
_SD = 128

_BB = 1024


def _fused_kernel(counts_ref, mask_ref, wblk_ref, bmap_ref, sel_ref,
                  wred_ref, bred_ref, out_ref):
    x = counts_ref[...]
    h = jnp.maximum(
        jnp.dot(x, wblk_ref[...], preferred_element_type=jnp.float32)
        + bmap_ref[...], 0.0)
    mk = jnp.dot(mask_ref[...].astype(jnp.float32), sel_ref[...],
                 preferred_element_type=jnp.float32)
    h = h * mk
    acc = jnp.dot(h, wred_ref[...],
                  preferred_element_type=jnp.float32) + bred_ref[...]
    out_ref[...] = jnp.maximum(acc, 0.0)


@jax.jit
def kernel(counts, observed_mask, W_map, b_map, W_red, b_red):
    counts2 = counts.reshape(_B, _NC * _CF)
    w_blk = jnp.kron(jnp.eye(_NC, dtype=W_map.dtype), W_map)
    b_tile = jnp.tile(b_map, _NC).reshape(1, _NC * _CD)
    sel = jnp.kron(jnp.eye(_NC, dtype=jnp.float32),
                   jnp.ones((1, _CD), jnp.float32))
    grid = _B // _BB
    return pl.pallas_call(
        _fused_kernel,
        grid=(grid,),
        in_specs=[
            pl.BlockSpec((_BB, _NC * _CF), lambda i: (i, 0)),
            pl.BlockSpec((_BB, _NC), lambda i: (i, 0)),
            pl.BlockSpec((_NC * _CF, _NC * _CD), lambda i: (0, 0)),
            pl.BlockSpec((1, _NC * _CD), lambda i: (0, 0)),
            pl.BlockSpec((_NC, _NC * _CD), lambda i: (0, 0)),
            pl.BlockSpec((_NC * _CD, _SD), lambda i: (0, 0)),
            pl.BlockSpec((1, _SD), lambda i: (0, 0)),
        ],
        out_specs=pl.BlockSpec((_BB, _SD), lambda i: (i, 0)),
        out_shape=jax.ShapeDtypeStruct((_B, _SD), jnp.float32),
    )(counts2, observed_mask, w_blk, b_tile, sel, W_red,
      b_red.reshape(1, _SD))


# speedup vs baseline: 2.5397x; 1.6210x over previous
import jax
import jax.numpy as jnp
from jax.experimental import pallas as pl

_B = 32768
_NC = 5
_CF = 64
_CD = 64
_SD = 128

_BB = 2048

_DN1 = (((0,), (0,)), ((), ()))
_DN2 = (((0,), (0,)), ((), ()))


def _fused_kernel(ct_ref, mask_ref, wmap_ref, bmap_ref, wred_ref,
                  bred_ref, out_ref):
    wm = wmap_ref[...]
    bm = bmap_ref[...]
    acc = jnp.zeros((out_ref.shape[0], out_ref.shape[1]), jnp.float32)
    for n in range(_NC):
        x = ct_ref[n]
        h = jnp.maximum(
            jax.lax.dot_general(wm, x, _DN1,
                                preferred_element_type=jnp.float32)
            + bm, 0.0)
        h = h * mask_ref[n:n + 1, :].astype(jnp.float32)
        acc = acc + jax.lax.dot_general(
            wred_ref[n], h, _DN2, preferred_element_type=jnp.float32)
    out_ref[...] = jnp.maximum(acc + bred_ref[...], 0.0)


@jax.jit
def kernel(counts, observed_mask, W_map, b_map, W_red, b_red):
    ct = jnp.transpose(counts, (1, 2, 0))
    mt = observed_mask.T
    wred3 = W_red.reshape(_NC, _CD, _SD)
    grid = _B // _BB
    outT = pl.pallas_call(
        _fused_kernel,
        grid=(grid,),
        in_specs=[
            pl.BlockSpec((_NC, _CF, _BB), lambda i: (0, 0, i)),
            pl.BlockSpec((_NC, _BB), lambda i: (0, i)),
            pl.BlockSpec((_CF, _CD), lambda i: (0, 0)),
            pl.BlockSpec((_CD, 1), lambda i: (0, 0)),
            pl.BlockSpec((_NC, _CD, _SD), lambda i: (0, 0, 0)),
            pl.BlockSpec((_SD, 1), lambda i: (0, 0)),
        ],
        out_specs=pl.BlockSpec((_SD, _BB), lambda i: (0, i)),
        out_shape=jax.ShapeDtypeStruct((_SD, _B), jnp.float32),
    )(ct, mt, W_map, b_map.reshape(_CD, 1), wred3, b_red.reshape(_SD, 1))
    return outT.T


# speedup vs baseline: 4.3211x; 1.7014x over previous
import jax
import jax.numpy as jnp
from jax.experimental import pallas as pl

_B = 32768
_NC = 5
_CF = 64
_CD = 64
_SD = 128

_BB = 2048

_DN1 = (((0,), (0,)), ((), ()))
_DN2 = (((0,), (0,)), ((), ()))


def _fused_kernel(ct_ref, mask_ref, wmap_ref, bmap_ref, wred_ref,
                  bred_ref, out_ref):
    wm = wmap_ref[...]
    bm = bmap_ref[...]
    acc = jnp.zeros((out_ref.shape[0], out_ref.shape[1]), jnp.float32)
    for n in range(_NC):
        x = ct_ref[n]
        h = jnp.maximum(
            jax.lax.dot_general(wm, x, _DN1,
                                preferred_element_type=jnp.float32)
            + bm, 0.0)
        h = h * mask_ref[n:n + 1, :].astype(jnp.float32)
        acc = acc + jax.lax.dot_general(
            h, wred_ref[n], _DN2, preferred_element_type=jnp.float32)
    out_ref[...] = jnp.maximum(acc + bred_ref[...], 0.0)


@jax.jit
def kernel(counts, observed_mask, W_map, b_map, W_red, b_red):
    ct = jnp.transpose(counts, (1, 2, 0))
    mt = observed_mask.T
    wred3 = W_red.reshape(_NC, _CD, _SD)
    grid = _B // _BB
    outT = pl.pallas_call(
        _fused_kernel,
        grid=(grid,),
        in_specs=[
            pl.BlockSpec((_NC, _CF, _BB), lambda i: (0, 0, i)),
            pl.BlockSpec((_NC, _BB), lambda i: (0, i)),
            pl.BlockSpec((_CF, _CD), lambda i: (0, 0)),
            pl.BlockSpec((_CD, 1), lambda i: (0, 0)),
            pl.BlockSpec((_NC, _CD, _SD), lambda i: (0, 0, 0)),
            pl.BlockSpec((1, _SD), lambda i: (0, 0)),
        ],
        out_specs=pl.BlockSpec((_BB, _SD), lambda i: (i, 0)),
        out_shape=jax.ShapeDtypeStruct((_B, _SD), jnp.float32),
    )(ct, mt, W_map, b_map.reshape(_CD, 1), wred3, b_red.reshape(1, _SD))
    return outT


# speedup vs baseline: 4.3826x; 1.0142x over previous
import jax
import jax.numpy as jnp
from jax.experimental import pallas as pl

_B = 32768
_NC = 5
_CF = 64
_CD = 64
_SD = 128

_BB = 2048

_DN1 = (((0,), (0,)), ((), ()))
_DN2 = (((0,), (0,)), ((), ()))


def _fused_kernel(ct_ref, mask_ref, wmap_ref, bmap_ref, wred_ref,
                  bred_ref, out_ref):
    wm = wmap_ref[...]
    bm = bmap_ref[...]
    acc = jnp.zeros((out_ref.shape[1], out_ref.shape[0]), jnp.float32)
    for n in range(_NC):
        x = ct_ref[n]
        h = jnp.maximum(
            jax.lax.dot_general(wm, x, _DN1,
                                preferred_element_type=jnp.float32)
            + bm, 0.0)
        h = h * mask_ref[n:n + 1, :].astype(jnp.float32)
        acc = acc + jax.lax.dot_general(
            wred_ref[n], h, _DN2, preferred_element_type=jnp.float32)
    out_ref[...] = jnp.maximum(acc + bred_ref[...], 0.0).T


@jax.jit
def kernel(counts, observed_mask, W_map, b_map, W_red, b_red):
    ct = jnp.transpose(counts, (1, 2, 0))
    mt = observed_mask.T
    wred3 = W_red.reshape(_NC, _CD, _SD)
    grid = _B // _BB
    outT = pl.pallas_call(
        _fused_kernel,
        grid=(grid,),
        in_specs=[
            pl.BlockSpec((_NC, _CF, _BB), lambda i: (0, 0, i)),
            pl.BlockSpec((_NC, _BB), lambda i: (0, i)),
            pl.BlockSpec((_CF, _CD), lambda i: (0, 0)),
            pl.BlockSpec((_CD, 1), lambda i: (0, 0)),
            pl.BlockSpec((_NC, _CD, _SD), lambda i: (0, 0, 0)),
            pl.BlockSpec((_SD, 1), lambda i: (0, 0)),
        ],
        out_specs=pl.BlockSpec((_BB, _SD), lambda i: (i, 0)),
        out_shape=jax.ShapeDtypeStruct((_B, _SD), jnp.float32),
    )(ct, mt, W_map, b_map.reshape(_CD, 1), wred3, b_red.reshape(_SD, 1))
    return outT


# speedup vs baseline: 5.1339x; 1.1714x over previous
import jax
import jax.numpy as jnp
from jax.experimental import pallas as pl

_B = 32768
_NC = 5
_CF = 64
_CD = 64
_SD = 128

_BB = 2048

_DN1 = (((0,), (0,)), ((), ()))
_DN2 = (((0,), (0,)), ((), ()))


def _fused_kernel(ct_ref, mask_ref, wmap_ref, bmap_ref, wred_ref,
                  bred_ref, out_ref):
    wm = wmap_ref[...]
    bm = bmap_ref[...]
    hs = []
    for n in range(_NC):
        x = ct_ref[n]
        h = jnp.maximum(
            jax.lax.dot_general(wm, x, _DN1,
                                preferred_element_type=jnp.float32)
            + bm, 0.0)
        hs.append(h * mask_ref[n:n + 1, :].astype(jnp.float32))
    hcat = jnp.concatenate(hs, axis=0)
    acc = jax.lax.dot_general(wred_ref[...], hcat, _DN2,
                              preferred_element_type=jnp.float32)
    out_ref[...] = jnp.maximum(acc + bred_ref[...], 0.0).T


@jax.jit
def kernel(counts, observed_mask, W_map, b_map, W_red, b_red):
    ct = jnp.transpose(counts, (1, 2, 0))
    mt = observed_mask.T
    grid = _B // _BB
    outT = pl.pallas_call(
        _fused_kernel,
        grid=(grid,),
        in_specs=[
            pl.BlockSpec((_NC, _CF, _BB), lambda i: (0, 0, i)),
            pl.BlockSpec((_NC, _BB), lambda i: (0, i)),
            pl.BlockSpec((_CF, _CD), lambda i: (0, 0)),
            pl.BlockSpec((_CD, 1), lambda i: (0, 0)),
            pl.BlockSpec((_NC * _CD, _SD), lambda i: (0, 0)),
            pl.BlockSpec((_SD, 1), lambda i: (0, 0)),
        ],
        out_specs=pl.BlockSpec((_BB, _SD), lambda i: (i, 0)),
        out_shape=jax.ShapeDtypeStruct((_B, _SD), jnp.float32),
    )(ct, mt, W_map, b_map.reshape(_CD, 1), W_red, b_red.reshape(_SD, 1))
    return outT


# speedup vs baseline: 6.2190x; 1.2113x over previous
import jax
import jax.numpy as jnp
from jax.experimental import pallas as pl

_B = 32768
_NC = 5
_CF = 64
_CD = 64
_SD = 128

_BB = 4096

_DN1 = (((0,), (0,)), ((), ()))
_DN2 = (((0,), (0,)), ((), ()))


def _fused_kernel(ct_ref, mask_ref, wmap_ref, bmap_ref, wred_ref,
                  bred_ref, out_ref):
    wm = wmap_ref[...]
    bm = bmap_ref[...]
    hs = []
    for n in range(_NC):
        x = ct_ref[n]
        h = jnp.maximum(
            jax.lax.dot_general(wm, x, _DN1,
                                preferred_element_type=jnp.float32)
            + bm, 0.0)
        hs.append(h * mask_ref[n:n + 1, :].astype(jnp.float32))
    hcat = jnp.concatenate(hs, axis=0)
    acc = jax.lax.dot_general(wred_ref[...], hcat, _DN2,
                              preferred_element_type=jnp.float32)
    out_ref[...] = jnp.maximum(acc + bred_ref[...], 0.0).T


@jax.jit
def kernel(counts, observed_mask, W_map, b_map, W_red, b_red):
    ct = jnp.transpose(counts, (1, 2, 0))
    mt = observed_mask.T
    grid = _B // _BB
    outT = pl.pallas_call(
        _fused_kernel,
        grid=(grid,),
        in_specs=[
            pl.BlockSpec((_NC, _CF, _BB), lambda i: (0, 0, i)),
            pl.BlockSpec((_NC, _BB), lambda i: (0, i)),
            pl.BlockSpec((_CF, _CD), lambda i: (0, 0)),
            pl.BlockSpec((_CD, 1), lambda i: (0, 0)),
            pl.BlockSpec((_NC * _CD, _SD), lambda i: (0, 0)),
            pl.BlockSpec((_SD, 1), lambda i: (0, 0)),
        ],
        out_specs=pl.BlockSpec((_BB, _SD), lambda i: (i, 0)),
        out_shape=jax.ShapeDtypeStruct((_B, _SD), jnp.float32),
    )(ct, mt, W_map, b_map.reshape(_CD, 1), W_red, b_red.reshape(_SD, 1))
    return outT


# speedup vs baseline: 6.4367x; 1.0350x over previous
import jax
import jax.numpy as jnp
from jax.experimental import pallas as pl

_B = 32768
_NC = 5
_CF = 64
_CD = 64
_SD = 128

_BB = 8192

_DN1 = (((0,), (0,)), ((), ()))
_DN2 = (((0,), (0,)), ((), ()))


def _fused_kernel(ct_ref, mask_ref, wmap_ref, bmap_ref, wred_ref,
                  bred_ref, out_ref):
    wm = wmap_ref[...]
    bm = bmap_ref[...]
    hs = []
    for n in range(_NC):
        x = ct_ref[n]
        h = jnp.maximum(
            jax.lax.dot_general(wm, x, _DN1,
                                preferred_element_type=jnp.float32)
            + bm, 0.0)
        hs.append(h * mask_ref[n:n + 1, :].astype(jnp.float32))
    hcat = jnp.concatenate(hs, axis=0)
    acc = jax.lax.dot_general(wred_ref[...], hcat, _DN2,
                              preferred_element_type=jnp.float32)
    out_ref[...] = jnp.maximum(acc + bred_ref[...], 0.0).T


@jax.jit
def kernel(counts, observed_mask, W_map, b_map, W_red, b_red):
    ct = jnp.transpose(counts, (1, 2, 0))
    mt = observed_mask.T
    grid = _B // _BB
    outT = pl.pallas_call(
        _fused_kernel,
        grid=(grid,),
        in_specs=[
            pl.BlockSpec((_NC, _CF, _BB), lambda i: (0, 0, i)),
            pl.BlockSpec((_NC, _BB), lambda i: (0, i)),
            pl.BlockSpec((_CF, _CD), lambda i: (0, 0)),
            pl.BlockSpec((_CD, 1), lambda i: (0, 0)),
            pl.BlockSpec((_NC * _CD, _SD), lambda i: (0, 0)),
            pl.BlockSpec((_SD, 1), lambda i: (0, 0)),
        ],
        out_specs=pl.BlockSpec((_BB, _SD), lambda i: (i, 0)),
        out_shape=jax.ShapeDtypeStruct((_B, _SD), jnp.float32),
    )(ct, mt, W_map, b_map.reshape(_CD, 1), W_red, b_red.reshape(_SD, 1))
    return outT
